# baseline (speedup 1.0000x reference)
import functools

import jax
import jax.numpy as jnp
from jax.experimental import pallas as pl

H = 256


def _pad_rows(x, bm):
    m = x.shape[0]
    mp = -(-m // bm) * bm
    if mp != m:
        x = jnp.pad(x, ((0, mp - m), (0, 0)))
    return x


def _mm_kernel(x_ref, w_ref, b_ref, o_ref, *, act):
    y = jnp.dot(x_ref[...], w_ref[...], preferred_element_type=jnp.float32)
    y = y + b_ref[...]
    if act == 'lrelu':
        y = jnp.where(y > 0, y, 0.01 * y)
    o_ref[...] = y


def _mm(x, w, b=None, act=None, bm=512):
    m, k = x.shape
    n = w.shape[1]
    if b is None:
        b = jnp.zeros((n,), jnp.float32)
    xp = _pad_rows(x, bm)
    grid = xp.shape[0] // bm
    out = pl.pallas_call(
        functools.partial(_mm_kernel, act=act),
        grid=(grid,),
        in_specs=[
            pl.BlockSpec((bm, k), lambda i: (i, 0)),
            pl.BlockSpec((k, n), lambda i: (0, 0)),
            pl.BlockSpec((1, n), lambda i: (0, 0)),
        ],
        out_specs=pl.BlockSpec((bm, n), lambda i: (i, 0)),
        out_shape=jax.ShapeDtypeStruct((xp.shape[0], n), jnp.float32),
    )(xp, w, b.reshape(1, n))
    return out[:m]


def _gru_kernel(x_ref, h_ref, wx_ref, wh_ref, bx_ref, bh_ref, o_ref):
    gx = jnp.dot(x_ref[...], wx_ref[...], preferred_element_type=jnp.float32)
    gx = gx + bx_ref[...]
    gh = jnp.dot(h_ref[...], wh_ref[...], preferred_element_type=jnp.float32)
    gh = gh + bh_ref[...]
    d = h_ref.shape[1]
    r = jax.nn.sigmoid(gx[:, :d] + gh[:, :d])
    z = jax.nn.sigmoid(gx[:, d:2 * d] + gh[:, d:2 * d])
    nn = jnp.tanh(gx[:, 2 * d:] + r * gh[:, 2 * d:])
    o_ref[...] = jnp.maximum((1 - z) * nn + z * h_ref[...], 0.0)


def _gru_relu(x, h, p, bm=512):
    m, d = x.shape
    xp = _pad_rows(x, bm)
    hp = _pad_rows(h, bm)
    grid = xp.shape[0] // bm
    out = pl.pallas_call(
        _gru_kernel,
        grid=(grid,),
        in_specs=[
            pl.BlockSpec((bm, d), lambda i: (i, 0)),
            pl.BlockSpec((bm, d), lambda i: (i, 0)),
            pl.BlockSpec((d, 3 * d), lambda i: (0, 0)),
            pl.BlockSpec((d, 3 * d), lambda i: (0, 0)),
            pl.BlockSpec((1, 3 * d), lambda i: (0, 0)),
            pl.BlockSpec((1, 3 * d), lambda i: (0, 0)),
        ],
        out_specs=pl.BlockSpec((bm, d), lambda i: (i, 0)),
        out_shape=jax.ShapeDtypeStruct((xp.shape[0], d), jnp.float32),
    )(xp, hp, p['Wx'], p['Wh'], p['bx'].reshape(1, -1), p['bh'].reshape(1, -1))
    return out[:m]


def _lrelu(x):
    return jnp.where(x > 0, x, 0.01 * x)


def _bn(x, g, b, eps=1e-5):
    mu = jnp.mean(x, 0)
    var = jnp.var(x, 0)
    return (x - mu) / jnp.sqrt(var + eps) * g + b


def _seg_softmax(a, seg, n):
    m = jax.ops.segment_max(a, seg, n)
    m = jnp.where(jnp.isfinite(m), m, 0.0)
    e = jnp.exp(a - m[seg])
    s = jax.ops.segment_sum(e, seg, n)
    return e / (s[seg] + 1e-9)


def _linbn(p, x):
    return _lrelu(_bn(_mm(x, p['W'], p['b']), p['g'], p['be']))


def _zeros(r, c):
    return jnp.zeros((r, c), jnp.float32)


def _atom_stage(heads, ei, h, e):
    src, dst = ei[0], ei[1]
    nh = len(heads)
    n = h.shape[0]
    msg = h[src] + e
    wbig = jnp.concatenate(
        [hp['Wc'] for hp in heads]
        + [hp['Wa'][H:] for hp in heads]
        + [_zeros(H, 128 - nh)], axis=1)
    big = _mm(msg, wbig)
    wa1 = jnp.concatenate([hp['Wa'][:H] for hp in heads]
                          + [_zeros(H, 128 - nh)], axis=1)
    u1 = _mm(h, wa1)[:, :nh]
    ba = jnp.stack([hp['ba'][0] for hp in heads])
    a = _lrelu(u1[dst] + big[:, nh * H:nh * H + nh] + ba)
    attn = _seg_softmax(a, dst, n)
    w3 = big[:, :nh * H].reshape(-1, nh, H) * attn[:, :, None]
    ctx = jax.nn.elu(jax.ops.segment_sum(w3, dst, n))
    return [_gru_relu(ctx[:, k], h, hp) for k, hp in enumerate(heads)]


def _mol_stage(ps, batch, hs_list, n):
    nh = len(ps)
    hcat = jnp.concatenate(hs_list, -1)
    hsum = jax.ops.segment_sum(hcat, batch, n)
    bigs = [_mm(hs_list[k],
                jnp.concatenate([ps[k]['Wc'], ps[k]['Wa'][H:],
                                 _zeros(H, 127)], axis=1))
            for k in range(nh)]
    u1 = jnp.stack(
        [_mm(hsum[:, k * H:(k + 1) * H],
             jnp.concatenate([ps[k]['Wa'][:H], _zeros(H, 127)], axis=1))[:, 0]
         for k in range(nh)], axis=1)
    ba = jnp.stack([p['ba'][0] for p in ps])
    a = _lrelu(u1[batch] + jnp.stack([b[:, H] for b in bigs], 1) + ba)
    attn = _seg_softmax(a, batch, n)
    w3 = jnp.stack([b[:, :H] for b in bigs], 1) * attn[:, :, None]
    ctx = jax.nn.elu(jax.ops.segment_sum(w3, batch, n))
    return [_gru_relu(ctx[:, k], hsum[:, k * H:(k + 1) * H], ps[k])
            for k in range(nh)]


def kernel(origin_node, origin_edge, frag_node, frag_edge, motif_node,
           motif_edge, params, origin_edge_index, origin_batch,
           frag_edge_index, frag_batch, motif_edge_index, motif_batch,
           channel_batch, index):
    nb = channel_batch.shape[0] // 3
    nm = motif_node.shape[0]

    on = _linbn(params['emb_node_o'], origin_node)
    oe = _linbn(params['emb_edge_o'], origin_edge)
    hns = _atom_stage([hp['atom'] for hp in params['origin_heads']],
                      origin_edge_index, on, oe)
    heads = _mol_stage([hp['mol'] for hp in params['origin_heads']],
                       origin_batch, hns, nb)
    p = params['origin_attend']
    graph_origin = jax.nn.relu(_bn(
        _mm(jnp.concatenate(heads, -1), p['W'], p['b']), p['g'], p['be']))

    fn = _linbn(params['emb_node_f'], frag_node)
    fe = _linbn(params['emb_edge_f'], frag_edge)
    hns = _atom_stage([hp['atom'] for hp in params['frag_heads']],
                      frag_edge_index, fn, fe)
    heads = _mol_stage([hp['mol'] for hp in params['frag_heads']],
                       frag_batch, hns, nm)
    p = params['frag_attend']
    graph_motif = jax.nn.relu(_bn(
        _mm(jnp.concatenate(heads, -1), p['W'], p['b']), p['g'], p['be']))
    motifs_series = jax.nn.relu(jax.ops.segment_sum(graph_motif,
                                                    motif_batch, nb))

    mn = _linbn(params['emb_frag_j'], motif_node)
    me = _linbn(params['emb_edge_j'], motif_edge)
    mnc = jnp.concatenate([graph_motif, mn], -1)
    hns = []
    for hp in params['junction_heads']:
        x = _mm(mnc, hp['proj_W'], hp['proj_b'])
        hns.append(_atom_stage([hp['atom']], motif_edge_index, x, me)[0])
    gh = _mol_stage([hp['mol'] for hp in params['junction_heads']],
                    motif_batch, hns, nb)
    super_new_graph = jax.nn.relu(jnp.mean(jnp.stack(gh, 1), 1))

    concat_features = jnp.concatenate(
        [graph_origin, super_new_graph, motifs_series], 0)
    super_hidden = _mol_stage([params['channel_mol']], channel_batch,
                              [concat_features], nb)[0]
    p = params['predict1']
    hpred = _bn(_mm(super_hidden, p['W'], p['b']), p['g'], p['be'])
    p2 = params['predict2']
    w2 = jnp.concatenate([p2['W'], _zeros(H, 127)], axis=1)
    b2 = jnp.concatenate([p2['b'], jnp.zeros((127,), jnp.float32)])
    out = _mm(_lrelu(hpred), w2, b2)[:, :1]
    return out[index]


# speedup vs baseline: 3.0227x; 3.0227x over previous
import functools

import jax
import jax.numpy as jnp
from jax.experimental import pallas as pl

H = 256


def _pad_rows(x, bm):
    m = x.shape[0]
    mp = -(-m // bm) * bm
    if mp != m:
        x = jnp.pad(x, ((0, mp - m), (0, 0)))
    return x


def _mm_kernel(x_ref, w_ref, b_ref, o_ref, *, act):
    y = jnp.dot(x_ref[...], w_ref[...], preferred_element_type=jnp.float32)
    y = y + b_ref[...]
    if act == 'lrelu':
        y = jnp.where(y > 0, y, 0.01 * y)
    o_ref[...] = y


def _mm(x, w, b=None, act=None, bm=512):
    m, k = x.shape
    n = w.shape[1]
    if b is None:
        b = jnp.zeros((n,), jnp.float32)
    xp = _pad_rows(x, bm)
    grid = xp.shape[0] // bm
    out = pl.pallas_call(
        functools.partial(_mm_kernel, act=act),
        grid=(grid,),
        in_specs=[
            pl.BlockSpec((bm, k), lambda i: (i, 0)),
            pl.BlockSpec((k, n), lambda i: (0, 0)),
            pl.BlockSpec((1, n), lambda i: (0, 0)),
        ],
        out_specs=pl.BlockSpec((bm, n), lambda i: (i, 0)),
        out_shape=jax.ShapeDtypeStruct((xp.shape[0], n), jnp.float32),
    )(xp, w, b.reshape(1, n))
    return out[:m]


def _gru_kernel(x_ref, h_ref, wx_ref, wh_ref, bx_ref, bh_ref, o_ref):
    gx = jnp.dot(x_ref[...], wx_ref[...], preferred_element_type=jnp.float32)
    gx = gx + bx_ref[...]
    gh = jnp.dot(h_ref[...], wh_ref[...], preferred_element_type=jnp.float32)
    gh = gh + bh_ref[...]
    d = h_ref.shape[1]
    r = jax.nn.sigmoid(gx[:, :d] + gh[:, :d])
    z = jax.nn.sigmoid(gx[:, d:2 * d] + gh[:, d:2 * d])
    nn = jnp.tanh(gx[:, 2 * d:] + r * gh[:, 2 * d:])
    o_ref[...] = jnp.maximum((1 - z) * nn + z * h_ref[...], 0.0)


def _gru_relu(x, h, p, bm=512):
    m, d = x.shape
    xp = _pad_rows(x, bm)
    hp = _pad_rows(h, bm)
    grid = xp.shape[0] // bm
    out = pl.pallas_call(
        _gru_kernel,
        grid=(grid,),
        in_specs=[
            pl.BlockSpec((bm, d), lambda i: (i, 0)),
            pl.BlockSpec((bm, d), lambda i: (i, 0)),
            pl.BlockSpec((d, 3 * d), lambda i: (0, 0)),
            pl.BlockSpec((d, 3 * d), lambda i: (0, 0)),
            pl.BlockSpec((1, 3 * d), lambda i: (0, 0)),
            pl.BlockSpec((1, 3 * d), lambda i: (0, 0)),
        ],
        out_specs=pl.BlockSpec((bm, d), lambda i: (i, 0)),
        out_shape=jax.ShapeDtypeStruct((xp.shape[0], d), jnp.float32),
    )(xp, hp, p['Wx'], p['Wh'], p['bx'].reshape(1, -1), p['bh'].reshape(1, -1))
    return out[:m]


def _lrelu(x):
    return jnp.where(x > 0, x, 0.01 * x)


def _bn(x, g, b, eps=1e-5):
    mu = jnp.mean(x, 0)
    var = jnp.var(x, 0)
    return (x - mu) / jnp.sqrt(var + eps) * g + b


def _seg_softmax(a, seg, n):
    m = jax.ops.segment_max(a, seg, n)
    m = jnp.where(jnp.isfinite(m), m, 0.0)
    e = jnp.exp(a - m[seg])
    s = jax.ops.segment_sum(e, seg, n)
    return e / (s[seg] + 1e-9)


def _linbn(p, x):
    return _lrelu(_bn(_mm(x, p['W'], p['b']), p['g'], p['be']))


def _zeros(r, c):
    return jnp.zeros((r, c), jnp.float32)


def _atom_stage(heads, ei, h, e):
    src, dst = ei[0], ei[1]
    nh = len(heads)
    n = h.shape[0]
    msg = h[src] + e
    wbig = jnp.concatenate(
        [hp['Wc'] for hp in heads]
        + [hp['Wa'][H:] for hp in heads]
        + [_zeros(H, 128 - nh)], axis=1)
    big = _mm(msg, wbig)
    wa1 = jnp.concatenate([hp['Wa'][:H] for hp in heads]
                          + [_zeros(H, 128 - nh)], axis=1)
    u1 = _mm(h, wa1)[:, :nh]
    ba = jnp.stack([hp['ba'][0] for hp in heads])
    a = _lrelu(u1[dst] + big[:, nh * H:nh * H + nh] + ba)
    attn = _seg_softmax(a, dst, n)
    outs = []
    for k, hp in enumerate(heads):
        ctx = jax.nn.elu(jax.ops.segment_sum(
            attn[:, k:k + 1] * big[:, k * H:(k + 1) * H], dst, n))
        outs.append(_gru_relu(ctx, h, hp))
    return outs


def _mol_stage(ps, batch, hs_list, n):
    nh = len(ps)
    hcat = jnp.concatenate(hs_list, -1)
    hsum = jax.ops.segment_sum(hcat, batch, n)
    bigs = [_mm(hs_list[k],
                jnp.concatenate([ps[k]['Wc'], ps[k]['Wa'][H:],
                                 _zeros(H, 127)], axis=1))
            for k in range(nh)]
    u1 = jnp.stack(
        [_mm(hsum[:, k * H:(k + 1) * H],
             jnp.concatenate([ps[k]['Wa'][:H], _zeros(H, 127)], axis=1))[:, 0]
         for k in range(nh)], axis=1)
    ba = jnp.stack([p['ba'][0] for p in ps])
    a = _lrelu(u1[batch] + jnp.stack([b[:, H] for b in bigs], 1) + ba)
    attn = _seg_softmax(a, batch, n)
    outs = []
    for k in range(nh):
        ctx = jax.nn.elu(jax.ops.segment_sum(
            attn[:, k:k + 1] * bigs[k][:, :H], batch, n))
        outs.append(_gru_relu(ctx, hsum[:, k * H:(k + 1) * H], ps[k]))
    return outs


def kernel(origin_node, origin_edge, frag_node, frag_edge, motif_node,
           motif_edge, params, origin_edge_index, origin_batch,
           frag_edge_index, frag_batch, motif_edge_index, motif_batch,
           channel_batch, index):
    nb = channel_batch.shape[0] // 3
    nm = motif_node.shape[0]

    on = _linbn(params['emb_node_o'], origin_node)
    oe = _linbn(params['emb_edge_o'], origin_edge)
    hns = _atom_stage([hp['atom'] for hp in params['origin_heads']],
                      origin_edge_index, on, oe)
    heads = _mol_stage([hp['mol'] for hp in params['origin_heads']],
                       origin_batch, hns, nb)
    p = params['origin_attend']
    graph_origin = jax.nn.relu(_bn(
        _mm(jnp.concatenate(heads, -1), p['W'], p['b']), p['g'], p['be']))

    fn = _linbn(params['emb_node_f'], frag_node)
    fe = _linbn(params['emb_edge_f'], frag_edge)
    hns = _atom_stage([hp['atom'] for hp in params['frag_heads']],
                      frag_edge_index, fn, fe)
    heads = _mol_stage([hp['mol'] for hp in params['frag_heads']],
                       frag_batch, hns, nm)
    p = params['frag_attend']
    graph_motif = jax.nn.relu(_bn(
        _mm(jnp.concatenate(heads, -1), p['W'], p['b']), p['g'], p['be']))
    motifs_series = jax.nn.relu(jax.ops.segment_sum(graph_motif,
                                                    motif_batch, nb))

    mn = _linbn(params['emb_frag_j'], motif_node)
    me = _linbn(params['emb_edge_j'], motif_edge)
    mnc = jnp.concatenate([graph_motif, mn], -1)
    hns = []
    for hp in params['junction_heads']:
        x = _mm(mnc, hp['proj_W'], hp['proj_b'])
        hns.append(_atom_stage([hp['atom']], motif_edge_index, x, me)[0])
    gh = _mol_stage([hp['mol'] for hp in params['junction_heads']],
                    motif_batch, hns, nb)
    super_new_graph = jax.nn.relu(jnp.mean(jnp.stack(gh, 1), 1))

    concat_features = jnp.concatenate(
        [graph_origin, super_new_graph, motifs_series], 0)
    super_hidden = _mol_stage([params['channel_mol']], channel_batch,
                              [concat_features], nb)[0]
    p = params['predict1']
    hpred = _bn(_mm(super_hidden, p['W'], p['b']), p['g'], p['be'])
    p2 = params['predict2']
    w2 = jnp.concatenate([p2['W'], _zeros(H, 127)], axis=1)
    b2 = jnp.concatenate([p2['b'], jnp.zeros((127,), jnp.float32)])
    out = _mm(_lrelu(hpred), w2, b2)[:, :1]
    return out[index]


# speedup vs baseline: 3.1500x; 1.0421x over previous
import functools

import jax
import jax.numpy as jnp
from jax.experimental import pallas as pl

H = 256


def _pad_rows(x, bm):
    m = x.shape[0]
    mp = -(-m // bm) * bm
    if mp != m:
        x = jnp.pad(x, ((0, mp - m), (0, 0)))
    return x


def _mm_kernel(x_ref, w_ref, b_ref, o_ref, *, act):
    y = jnp.dot(x_ref[...], w_ref[...], preferred_element_type=jnp.float32)
    y = y + b_ref[...]
    if act == 'lrelu':
        y = jnp.where(y > 0, y, 0.01 * y)
    o_ref[...] = y


def _mm(x, w, b=None, act=None, bm=512):
    m, k = x.shape
    n = w.shape[1]
    if b is None:
        b = jnp.zeros((n,), jnp.float32)
    xp = _pad_rows(x, bm)
    grid = xp.shape[0] // bm
    out = pl.pallas_call(
        functools.partial(_mm_kernel, act=act),
        grid=(grid,),
        in_specs=[
            pl.BlockSpec((bm, k), lambda i: (i, 0)),
            pl.BlockSpec((k, n), lambda i: (0, 0)),
            pl.BlockSpec((1, n), lambda i: (0, 0)),
        ],
        out_specs=pl.BlockSpec((bm, n), lambda i: (i, 0)),
        out_shape=jax.ShapeDtypeStruct((xp.shape[0], n), jnp.float32),
    )(xp, w, b.reshape(1, n))
    return out[:m]


def _gru_kernel(x_ref, h_ref, wx_ref, wh_ref, bx_ref, bh_ref, o_ref):
    gx = jnp.dot(x_ref[...], wx_ref[...], preferred_element_type=jnp.float32)
    gx = gx + bx_ref[...]
    gh = jnp.dot(h_ref[...], wh_ref[...], preferred_element_type=jnp.float32)
    gh = gh + bh_ref[...]
    d = h_ref.shape[1]
    r = jax.nn.sigmoid(gx[:, :d] + gh[:, :d])
    z = jax.nn.sigmoid(gx[:, d:2 * d] + gh[:, d:2 * d])
    nn = jnp.tanh(gx[:, 2 * d:] + r * gh[:, 2 * d:])
    o_ref[...] = jnp.maximum((1 - z) * nn + z * h_ref[...], 0.0)


def _gru_relu(x, h, p, bm=512):
    m, d = x.shape
    xp = _pad_rows(x, bm)
    hp = _pad_rows(h, bm)
    grid = xp.shape[0] // bm
    out = pl.pallas_call(
        _gru_kernel,
        grid=(grid,),
        in_specs=[
            pl.BlockSpec((bm, d), lambda i: (i, 0)),
            pl.BlockSpec((bm, d), lambda i: (i, 0)),
            pl.BlockSpec((d, 3 * d), lambda i: (0, 0)),
            pl.BlockSpec((d, 3 * d), lambda i: (0, 0)),
            pl.BlockSpec((1, 3 * d), lambda i: (0, 0)),
            pl.BlockSpec((1, 3 * d), lambda i: (0, 0)),
        ],
        out_specs=pl.BlockSpec((bm, d), lambda i: (i, 0)),
        out_shape=jax.ShapeDtypeStruct((xp.shape[0], d), jnp.float32),
    )(xp, hp, p['Wx'], p['Wh'], p['bx'].reshape(1, -1), p['bh'].reshape(1, -1))
    return out[:m]


def _lrelu(x):
    return jnp.where(x > 0, x, 0.01 * x)


def _bn(x, g, b, eps=1e-5):
    mu = jnp.mean(x, 0)
    var = jnp.var(x, 0)
    return (x - mu) / jnp.sqrt(var + eps) * g + b


def _seg_softmax(a, seg, n):
    m = jax.ops.segment_max(a, seg, n)
    m = jnp.where(jnp.isfinite(m), m, 0.0)
    e = jnp.exp(a - m[seg])
    s = jax.ops.segment_sum(e, seg, n)
    return e / (s[seg] + 1e-9)


def _linbn(p, x):
    return _lrelu(_bn(_mm(x, p['W'], p['b']), p['g'], p['be']))


def _zeros(r, c):
    return jnp.zeros((r, c), jnp.float32)


def _atom_stage(heads, ei, h, e):
    src, dst = ei[0], ei[1]
    nh = len(heads)
    n = h.shape[0]
    msg = h[src] + e
    wbig = jnp.concatenate(
        [hp['Wc'] for hp in heads]
        + [hp['Wa'][H:] for hp in heads]
        + [_zeros(H, 128 - nh)], axis=1)
    big = _mm(msg, wbig)
    wa1 = jnp.concatenate([hp['Wa'][:H] for hp in heads]
                          + [_zeros(H, 128 - nh)], axis=1)
    u1 = _mm(h, wa1)[:, :nh]
    ba = jnp.stack([hp['ba'][0] for hp in heads])
    a = _lrelu(u1[dst] + big[:, nh * H:nh * H + nh] + ba)
    attn = _seg_softmax(a, dst, n)
    outs = []
    for k, hp in enumerate(heads):
        wmsg = (attn[:, k:k + 1] * big[:, k * H:(k + 1) * H])
        ctx = jax.nn.elu(jax.ops.segment_sum(
            wmsg.astype(jnp.bfloat16), dst, n).astype(jnp.float32))
        outs.append(_gru_relu(ctx, h, hp))
    return outs


def _mol_stage(ps, batch, hs_list, n):
    nh = len(ps)
    hcat = jnp.concatenate(hs_list, -1)
    hsum = jax.ops.segment_sum(hcat, batch, n)
    bigs = [_mm(hs_list[k],
                jnp.concatenate([ps[k]['Wc'], ps[k]['Wa'][H:],
                                 _zeros(H, 127)], axis=1))
            for k in range(nh)]
    u1 = jnp.stack(
        [_mm(hsum[:, k * H:(k + 1) * H],
             jnp.concatenate([ps[k]['Wa'][:H], _zeros(H, 127)], axis=1))[:, 0]
         for k in range(nh)], axis=1)
    ba = jnp.stack([p['ba'][0] for p in ps])
    a = _lrelu(u1[batch] + jnp.stack([b[:, H] for b in bigs], 1) + ba)
    attn = _seg_softmax(a, batch, n)
    outs = []
    for k in range(nh):
        ctx = jax.nn.elu(jax.ops.segment_sum(
            attn[:, k:k + 1] * bigs[k][:, :H], batch, n))
        outs.append(_gru_relu(ctx, hsum[:, k * H:(k + 1) * H], ps[k]))
    return outs


def kernel(origin_node, origin_edge, frag_node, frag_edge, motif_node,
           motif_edge, params, origin_edge_index, origin_batch,
           frag_edge_index, frag_batch, motif_edge_index, motif_batch,
           channel_batch, index):
    nb = channel_batch.shape[0] // 3
    nm = motif_node.shape[0]

    on = _linbn(params['emb_node_o'], origin_node)
    oe = _linbn(params['emb_edge_o'], origin_edge)
    hns = _atom_stage([hp['atom'] for hp in params['origin_heads']],
                      origin_edge_index, on, oe)
    heads = _mol_stage([hp['mol'] for hp in params['origin_heads']],
                       origin_batch, hns, nb)
    p = params['origin_attend']
    graph_origin = jax.nn.relu(_bn(
        _mm(jnp.concatenate(heads, -1), p['W'], p['b']), p['g'], p['be']))

    fn = _linbn(params['emb_node_f'], frag_node)
    fe = _linbn(params['emb_edge_f'], frag_edge)
    hns = _atom_stage([hp['atom'] for hp in params['frag_heads']],
                      frag_edge_index, fn, fe)
    heads = _mol_stage([hp['mol'] for hp in params['frag_heads']],
                       frag_batch, hns, nm)
    p = params['frag_attend']
    graph_motif = jax.nn.relu(_bn(
        _mm(jnp.concatenate(heads, -1), p['W'], p['b']), p['g'], p['be']))
    motifs_series = jax.nn.relu(jax.ops.segment_sum(graph_motif,
                                                    motif_batch, nb))

    mn = _linbn(params['emb_frag_j'], motif_node)
    me = _linbn(params['emb_edge_j'], motif_edge)
    mnc = jnp.concatenate([graph_motif, mn], -1)
    hns = []
    for hp in params['junction_heads']:
        x = _mm(mnc, hp['proj_W'], hp['proj_b'])
        hns.append(_atom_stage([hp['atom']], motif_edge_index, x, me)[0])
    gh = _mol_stage([hp['mol'] for hp in params['junction_heads']],
                    motif_batch, hns, nb)
    super_new_graph = jax.nn.relu(jnp.mean(jnp.stack(gh, 1), 1))

    concat_features = jnp.concatenate(
        [graph_origin, super_new_graph, motifs_series], 0)
    super_hidden = _mol_stage([params['channel_mol']], channel_batch,
                              [concat_features], nb)[0]
    p = params['predict1']
    hpred = _bn(_mm(super_hidden, p['W'], p['b']), p['g'], p['be'])
    p2 = params['predict2']
    w2 = jnp.concatenate([p2['W'], _zeros(H, 127)], axis=1)
    b2 = jnp.concatenate([p2['b'], jnp.zeros((127,), jnp.float32)])
    out = _mm(_lrelu(hpred), w2, b2)[:, :1]
    return out[index]
